# baseline (speedup 1.0000x reference)
import functools

import jax
import jax.numpy as jnp
from jax import lax
from jax.experimental import pallas as pl
from jax.experimental.pallas import tpu as pltpu
from jax.experimental.pallas import tpu_sc as plsc

N, K, D, H, L = 10000, 16, 128, 128, 3
EPS = 1e-5
SCALE = 30.0

NB = 400
EB = NB * K
GRID = N // NB

B = N * K
B_PAD = 163840
IDX_ROWS = B_PAD // 128
NWORKERS = 32
CHUNKS_PER_W = IDX_ROWS // NWORKERS


def _gelu(x):
    return 0.5 * x * (1.0 + lax.erf(x * 0.7071067811865476))


def _bdot(a, b):
    return jnp.dot(a.astype(jnp.bfloat16), b.astype(jnp.bfloat16),
                   preferred_element_type=jnp.float32)


def _ln(x, w, b):
    m = jnp.mean(x, axis=-1, keepdims=True)
    xc = x - m
    v = jnp.mean(xc * xc, axis=-1, keepdims=True)
    return xc * lax.rsqrt(v + EPS) * w + b


def _tile_nodes(x):
    return jnp.broadcast_to(x[:, None, :], (NB, K, x.shape[-1])).reshape(EB, x.shape[-1])


def _msg_sum(m):
    return jnp.sum(m.reshape(NB, K, D), axis=1)


def _node_update(node, msum, ln1w, ln1b, dwinT, dbin, dwoutT, dbout, ln2w, ln2b, mask):
    nd = _ln(node + msum * (1.0 / SCALE), ln1w, ln1b)
    d_ = _bdot(_gelu(_bdot(nd, dwinT) + dbin), dwoutT) + dbout
    return _ln(nd + d_, ln2w, ln2b) * mask


def _init_body(ef_ref, mask_ref,
               w1mT, b1, w2T, b2, w3T, b3,
               ln1w, ln1b, dwinT, dbin, dwoutT, dbout, ln2w, ln2b,
               node_out):
    ef = ef_ref[...]
    h = _gelu(_bdot(ef, w1mT[...]) + b1[...])
    h = _gelu(_bdot(h, w2T[...]) + b2[...])
    m = _bdot(h, w3T[...]) + b3[...]
    node_out[...] = _node_update(
        jnp.zeros((NB, D), jnp.float32), _msg_sum(m),
        ln1w[...], ln1b[...], dwinT[...], dbin[...], dwoutT[...], dbout[...],
        ln2w[...], ln2b[...], mask_ref[...])


def _edge_mlp(node, ef, nbr, waT, wbT, wcT, b1, w2T, b2, w3T, b3):
    hn = _bdot(node, waT)
    h = _gelu(_tile_nodes(hn)
              + _bdot(ef, wbT)
              + _bdot(nbr, wcT) + b1)
    h = _gelu(_bdot(h, w2T) + b2)
    return _bdot(h, w3T) + b3


def _fused_body(ef_ref, nbr_ref, node_ref, mask_ref,
                w11aT, w11bT, w11cT, b11, w12T, b12, w13T, b13, ln3w, ln3b,
                w1aT, w1bT, w1cT, b1, w2T, b2, w3T, b3,
                ln1w, ln1b, dwinT, dbin, dwoutT, dbout, ln2w, ln2b,
                ef_out, node_out):
    ef = ef_ref[...]
    nbr = nbr_ref[...]
    node = node_ref[...]
    em = _edge_mlp(node, ef, nbr, w11aT[...], w11bT[...], w11cT[...],
                   b11[...], w12T[...], b12[...], w13T[...], b13[...])
    efn = _ln(ef + em, ln3w[...], ln3b[...])
    ef_out[...] = efn
    m = _edge_mlp(node, efn, nbr, w1aT[...], w1bT[...], w1cT[...],
                  b1[...], w2T[...], b2[...], w3T[...], b3[...])
    node_out[...] = _node_update(
        node, _msg_sum(m),
        ln1w[...], ln1b[...], dwinT[...], dbin[...], dwoutT[...], dbout[...],
        ln2w[...], ln2b[...], mask_ref[...])


def _final_body(ef_ref, nbr_ref, node_ref,
                w11aT, w11bT, w11cT, b11, w12T, b12, w13T, b13, ln3w, ln3b,
                ef_out):
    ef = ef_ref[...]
    em = _edge_mlp(node_ref[...], ef, nbr_ref[...],
                   w11aT[...], w11bT[...], w11cT[...],
                   b11[...], w12T[...], b12[...], w13T[...], b13[...])
    ef_out[...] = _ln(ef + em, ln3w[...], ln3b[...])


_EDGE_SPEC = pl.BlockSpec((EB, D), lambda i: (i, 0))
_NBR_SPEC = pl.BlockSpec((EB, D), lambda i: (i, 0))
_NODE_SPEC = pl.BlockSpec((NB, D), lambda i: (i, 0))
_MASK_SPEC = pl.BlockSpec((NB, 1), lambda i: (i, 0))


def _w_spec(shape):
    return pl.BlockSpec(shape, lambda i: tuple(0 for _ in shape))


def _wspecs(shapes):
    return [_w_spec(s) for s in shapes]


_MAT = (D, H)
_VEC = (1, H)
_MLP_W_SHAPES = [_MAT, _MAT, _MAT, _VEC, _MAT, _VEC, _MAT, _VEC]
_LN_SHAPES = [_VEC, _VEC]
_FFN_SHAPES = [_VEC, _VEC, _MAT, _VEC, _MAT, _VEC, _VEC, _VEC]

_TC_PARAMS = pltpu.CompilerParams(dimension_semantics=("arbitrary",))


def _call_init(ef2d, mask2d, *weights):
    return pl.pallas_call(
        _init_body,
        grid=(GRID,),
        in_specs=[_EDGE_SPEC, _MASK_SPEC]
        + _wspecs([_MAT, _VEC, _MAT, _VEC, _MAT, _VEC] + _FFN_SHAPES),
        out_specs=_NODE_SPEC,
        out_shape=jax.ShapeDtypeStruct((N, D), jnp.float32),
        compiler_params=_TC_PARAMS,
    )(ef2d, mask2d, *weights)


def _call_fused(ef2d, nbr, node, mask2d, *weights):
    return pl.pallas_call(
        _fused_body,
        grid=(GRID,),
        in_specs=[_EDGE_SPEC, _NBR_SPEC, _NODE_SPEC, _MASK_SPEC]
        + _wspecs(_MLP_W_SHAPES + _LN_SHAPES + _MLP_W_SHAPES + _FFN_SHAPES),
        out_specs=[_EDGE_SPEC, _NODE_SPEC],
        out_shape=[jax.ShapeDtypeStruct((B, D), jnp.float32),
                   jax.ShapeDtypeStruct((N, D), jnp.float32)],
        compiler_params=_TC_PARAMS,
    )(ef2d, nbr, node, mask2d, *weights)


def _call_final(ef2d, nbr, node, *weights):
    return pl.pallas_call(
        _final_body,
        grid=(GRID,),
        in_specs=[_EDGE_SPEC, _NBR_SPEC, _NODE_SPEC]
        + _wspecs(_MLP_W_SHAPES + _LN_SHAPES),
        out_specs=_EDGE_SPEC,
        out_shape=jax.ShapeDtypeStruct((B, D), jnp.float32),
        compiler_params=_TC_PARAMS,
    )(ef2d, nbr, node, *weights)


_NBUF = 4
_PAIR = 2 * CHUNKS_PER_W
_C_FAST = 64
_C_SLOW = _PAIR - _C_FAST
_FAST_CORE = 0


def _sc_gather(node_table, idx2d):
    mesh = plsc.VectorSubcoreMesh(core_axis_name="c", subcore_axis_name="s")

    @functools.partial(
        pl.kernel,
        out_type=jax.ShapeDtypeStruct((B_PAD, D), jnp.float32),
        mesh=mesh,
        scratch_types=[
            pltpu.VMEM((_PAIR, 128), jnp.int32),
            [pltpu.VMEM((128, D), jnp.float32) for _ in range(_NBUF)],
            [pltpu.SemaphoreType.DMA for _ in range(_NBUF)],
            [pltpu.SemaphoreType.DMA for _ in range(_NBUF)],
        ],
    )
    def gk(table_hbm, idx_hbm, out_hbm, idx_v, bufs, gsems, ssems):
        s = lax.axis_index("s")
        c = lax.axis_index("c")
        is_fast = (c == _FAST_CORE)
        loff = jnp.where(is_fast, 0, _C_FAST)
        ngroups = jnp.where(is_fast, _C_FAST // _NBUF, _C_SLOW // _NBUF)
        pbase = s * _PAIR
        pltpu.sync_copy(idx_hbm.at[pl.ds(pbase, _PAIR)], idx_v)

        for b in range(_NBUF):
            pltpu.async_copy(table_hbm.at[idx_v.at[loff + b]], bufs[b],
                             gsems[b])

        def body(g, carry):
            for b in range(_NBUF):
                i = loff + g * _NBUF + b
                pltpu.make_async_copy(
                    table_hbm.at[idx_v.at[i]], bufs[b], gsems[b]).wait()
                pltpu.async_copy(
                    bufs[b], out_hbm.at[pl.ds((pbase + i) * 128, 128)],
                    ssems[b])

                @pl.when(g < ngroups - 1)
                def _():
                    pltpu.make_async_copy(
                        bufs[b], out_hbm.at[pl.ds(pbase * 128, 128)],
                        ssems[b]).wait()
                    pltpu.async_copy(
                        table_hbm.at[idx_v.at[i + _NBUF]], bufs[b], gsems[b])
            return carry

        lax.fori_loop(0, ngroups, body, 0)
        for b in range(_NBUF):
            pltpu.make_async_copy(
                bufs[b], out_hbm.at[pl.ds(pbase * 128, 128)], ssems[b]).wait()

    return gk(node_table, idx2d)


def _mlp_weights(l, Wa, Ba, W2, B2, W3, B3):
    w = Wa[l]
    return (w[:, :D].T, w[:, D:2 * D].T, w[:, 2 * D:].T, Ba[l][None, :],
            W2[l].T, B2[l][None, :], W3[l].T, B3[l][None, :])


def _ffn_weights(l, LN1w, LN1b, DWin, DBin, DWout, DBout, LN2w, LN2b):
    return (LN1w[l][None, :], LN1b[l][None, :], DWin[l].T, DBin[l][None, :],
            DWout[l].T, DBout[l][None, :], LN2w[l][None, :], LN2b[l][None, :])


def kernel(edge_features, neighbor_indices, mask, W1, B1, W2, B2, W3, B3,
           LN1w, LN1b, DWin, DBin, DWout, DBout, LN2w, LN2b,
           W11, B11, W12, B12, W13, B13, LN3w, LN3b):
    ef2d = edge_features.reshape(B, D)
    mask2d = mask[:, None]
    idx = neighbor_indices.reshape(-1).astype(jnp.int32)
    idx2d = jnp.concatenate(
        [idx, jnp.zeros((B_PAD - B,), jnp.int32)]).reshape(IDX_ROWS, 128)

    init_w = ((W1[0][:, D:2 * D].T, B1[0][None, :], W2[0].T, B2[0][None, :],
               W3[0].T, B3[0][None, :])
              + _ffn_weights(0, LN1w, LN1b, DWin, DBin, DWout, DBout, LN2w, LN2b))
    node = _call_init(ef2d, mask2d, *init_w)

    for l in range(L - 1):
        nbr = _sc_gather(node, idx2d)
        w = (_mlp_weights(l, W11, B11, W12, B12, W13, B13)
             + (LN3w[l][None, :], LN3b[l][None, :])
             + _mlp_weights(l + 1, W1, B1, W2, B2, W3, B3)
             + _ffn_weights(l + 1, LN1w, LN1b, DWin, DBin, DWout, DBout,
                            LN2w, LN2b))
        ef2d, node = _call_fused(ef2d, nbr, node, mask2d, *w)

    nbr = _sc_gather(node, idx2d)
    w = (_mlp_weights(L - 1, W11, B11, W12, B12, W13, B13)
         + (LN3w[L - 1][None, :], LN3b[L - 1][None, :]))
    ef2d = _call_final(ef2d, nbr, node, *w)

    return node, ef2d.reshape(N, K, D)


# speedup vs baseline: 1.0359x; 1.0359x over previous
import functools

import jax
import jax.numpy as jnp
from jax import lax
from jax.experimental import pallas as pl
from jax.experimental.pallas import tpu as pltpu
from jax.experimental.pallas import tpu_sc as plsc

N, K, D, H, L = 10000, 16, 128, 128, 3
EPS = 1e-5
SCALE = 30.0

NB = 400
EB = NB * K
GRID = N // NB

B = N * K
B_PAD = 163840
IDX_ROWS = B_PAD // 128
NWORKERS = 32
CHUNKS_PER_W = IDX_ROWS // NWORKERS


def _gelu(x):
    return x * (lax.erf(x * 0.7071067811865476) * 0.5 + 0.5)


def _bdot(a, b):
    return jnp.dot(a.astype(jnp.bfloat16), b.astype(jnp.bfloat16),
                   preferred_element_type=jnp.float32)


def _ln(x, w, b):
    m = jnp.mean(x, axis=-1, keepdims=True)
    xc = x - m
    v = jnp.mean(xc * xc, axis=-1, keepdims=True)
    return xc * lax.rsqrt(v + EPS) * w + b


def _tile_nodes(x):
    return jnp.broadcast_to(x[:, None, :], (NB, K, x.shape[-1])).reshape(EB, x.shape[-1])


def _msg_sum(m):
    return jnp.sum(m.reshape(NB, K, m.shape[-1]), axis=1)


def _node_update(node, msum, ln1w, ln1b, dwinT, dbin, dwoutT, dbout, ln2w, ln2b, mask):
    nd = _ln(node + msum * (1.0 / SCALE), ln1w, ln1b)
    d_ = _bdot(_gelu(_bdot(nd, dwinT) + dbin), dwoutT) + dbout
    return _ln(nd + d_, ln2w, ln2b) * mask


def _init_body(ef_ref, mask_ref,
               w1mT, b1, w2T, b2, w3T, b3,
               ln1w, ln1b, dwinT, dbin, dwoutT, dbout, ln2w, ln2b,
               node_out):
    ef = ef_ref[...]
    h = _gelu(_bdot(ef, w1mT[...]) + b1[...])
    h = _gelu(_bdot(h, w2T[...]) + b2[...])
    msum = _bdot(_msg_sum(h), w3T[...]) + float(K) * b3[...]
    node_out[...] = _node_update(
        jnp.zeros((NB, D), jnp.float32), msum,
        ln1w[...], ln1b[...], dwinT[...], dbin[...], dwoutT[...], dbout[...],
        ln2w[...], ln2b[...], mask_ref[...])


def _mlp2(node, ef, nbr, waT, wbT, wcT, b1, w2T, b2):
    hn = _bdot(node, waT)
    h = _gelu(_tile_nodes(hn)
              + _bdot(ef, wbT)
              + _bdot(nbr, wcT) + b1)
    return _gelu(_bdot(h, w2T) + b2)


def _fused_body(ef_ref, nbr_ref, node_ref, mask_ref,
                w11aT, w11bT, w11cT, b11, w12T, b12, w13T, b13, ln3w, ln3b,
                w1aT, w1bT, w1cT, b1, w2T, b2, w3T, b3,
                ln1w, ln1b, dwinT, dbin, dwoutT, dbout, ln2w, ln2b,
                ef_out, node_out):
    ef = ef_ref[...]
    nbr = nbr_ref[...]
    node = node_ref[...]
    h = _mlp2(node, ef, nbr, w11aT[...], w11bT[...], w11cT[...],
              b11[...], w12T[...], b12[...])
    em = _bdot(h, w13T[...]) + b13[...]
    efn = _ln(ef + em, ln3w[...], ln3b[...])
    ef_out[...] = efn
    h = _mlp2(node, efn, nbr, w1aT[...], w1bT[...], w1cT[...],
              b1[...], w2T[...], b2[...])
    msum = _bdot(_msg_sum(h), w3T[...]) + float(K) * b3[...]
    node_out[...] = _node_update(
        node, msum,
        ln1w[...], ln1b[...], dwinT[...], dbin[...], dwoutT[...], dbout[...],
        ln2w[...], ln2b[...], mask_ref[...])


def _final_body(ef_ref, nbr_ref, node_ref,
                w11aT, w11bT, w11cT, b11, w12T, b12, w13T, b13, ln3w, ln3b,
                ef_out):
    ef = ef_ref[...]
    h = _mlp2(node_ref[...], ef, nbr_ref[...],
              w11aT[...], w11bT[...], w11cT[...],
              b11[...], w12T[...], b12[...])
    em = _bdot(h, w13T[...]) + b13[...]
    ef_out[...] = _ln(ef + em, ln3w[...], ln3b[...])


_EDGE_SPEC = pl.BlockSpec((EB, D), lambda i: (i, 0))
_NBR_SPEC = pl.BlockSpec((EB, D), lambda i: (i, 0))
_NODE_SPEC = pl.BlockSpec((NB, D), lambda i: (i, 0))
_MASK_SPEC = pl.BlockSpec((NB, 1), lambda i: (i, 0))


def _w_spec(shape):
    return pl.BlockSpec(shape, lambda i: tuple(0 for _ in shape))


def _wspecs(shapes):
    return [_w_spec(s) for s in shapes]


_MAT = (D, H)
_VEC = (1, H)
_MLP_W_SHAPES = [_MAT, _MAT, _MAT, _VEC, _MAT, _VEC, _MAT, _VEC]
_LN_SHAPES = [_VEC, _VEC]
_FFN_SHAPES = [_VEC, _VEC, _MAT, _VEC, _MAT, _VEC, _VEC, _VEC]

_TC_PARAMS = pltpu.CompilerParams(dimension_semantics=("arbitrary",))


def _call_init(ef2d, mask2d, *weights):
    return pl.pallas_call(
        _init_body,
        grid=(GRID,),
        in_specs=[_EDGE_SPEC, _MASK_SPEC]
        + _wspecs([_MAT, _VEC, _MAT, _VEC, _MAT, _VEC] + _FFN_SHAPES),
        out_specs=_NODE_SPEC,
        out_shape=jax.ShapeDtypeStruct((N, D), jnp.float32),
        compiler_params=_TC_PARAMS,
    )(ef2d, mask2d, *weights)


def _call_fused(ef2d, nbr, node, mask2d, *weights):
    return pl.pallas_call(
        _fused_body,
        grid=(GRID,),
        in_specs=[_EDGE_SPEC, _NBR_SPEC, _NODE_SPEC, _MASK_SPEC]
        + _wspecs(_MLP_W_SHAPES + _LN_SHAPES + _MLP_W_SHAPES + _FFN_SHAPES),
        out_specs=[_EDGE_SPEC, _NODE_SPEC],
        out_shape=[jax.ShapeDtypeStruct((B, D), jnp.float32),
                   jax.ShapeDtypeStruct((N, D), jnp.float32)],
        compiler_params=_TC_PARAMS,
    )(ef2d, nbr, node, mask2d, *weights)


def _call_final(ef2d, nbr, node, *weights):
    return pl.pallas_call(
        _final_body,
        grid=(GRID,),
        in_specs=[_EDGE_SPEC, _NBR_SPEC, _NODE_SPEC]
        + _wspecs(_MLP_W_SHAPES + _LN_SHAPES),
        out_specs=_EDGE_SPEC,
        out_shape=jax.ShapeDtypeStruct((B, D), jnp.float32),
        compiler_params=_TC_PARAMS,
    )(ef2d, nbr, node, *weights)


_NBUF = 4
_PAIR = 2 * CHUNKS_PER_W
_C_FAST = 64
_C_SLOW = _PAIR - _C_FAST
_FAST_CORE = 0


def _sc_gather(node_table, idx2d):
    mesh = plsc.VectorSubcoreMesh(core_axis_name="c", subcore_axis_name="s")

    @functools.partial(
        pl.kernel,
        out_type=jax.ShapeDtypeStruct((B_PAD, D), jnp.float32),
        mesh=mesh,
        scratch_types=[
            pltpu.VMEM((_PAIR, 128), jnp.int32),
            [pltpu.VMEM((128, D), jnp.float32) for _ in range(_NBUF)],
            [pltpu.SemaphoreType.DMA for _ in range(_NBUF)],
            [pltpu.SemaphoreType.DMA for _ in range(_NBUF)],
        ],
    )
    def gk(table_hbm, idx_hbm, out_hbm, idx_v, bufs, gsems, ssems):
        s = lax.axis_index("s")
        c = lax.axis_index("c")
        is_fast = (c == _FAST_CORE)
        loff = jnp.where(is_fast, 0, _C_FAST)
        ngroups = jnp.where(is_fast, _C_FAST // _NBUF, _C_SLOW // _NBUF)
        pbase = s * _PAIR
        pltpu.sync_copy(idx_hbm.at[pl.ds(pbase, _PAIR)], idx_v)

        for b in range(_NBUF):
            pltpu.async_copy(table_hbm.at[idx_v.at[loff + b]], bufs[b],
                             gsems[b])

        def body(g, carry):
            for b in range(_NBUF):
                i = loff + g * _NBUF + b
                pltpu.make_async_copy(
                    table_hbm.at[idx_v.at[i]], bufs[b], gsems[b]).wait()
                pltpu.async_copy(
                    bufs[b], out_hbm.at[pl.ds((pbase + i) * 128, 128)],
                    ssems[b])

                @pl.when(g < ngroups - 1)
                def _():
                    pltpu.make_async_copy(
                        bufs[b], out_hbm.at[pl.ds(pbase * 128, 128)],
                        ssems[b]).wait()
                    pltpu.async_copy(
                        table_hbm.at[idx_v.at[i + _NBUF]], bufs[b], gsems[b])
            return carry

        lax.fori_loop(0, ngroups, body, 0)
        for b in range(_NBUF):
            pltpu.make_async_copy(
                bufs[b], out_hbm.at[pl.ds(pbase * 128, 128)], ssems[b]).wait()

    return gk(node_table, idx2d)


def _mlp_weights(l, Wa, Ba, W2, B2, W3, B3):
    w = Wa[l]
    return (w[:, :D].T, w[:, D:2 * D].T, w[:, 2 * D:].T, Ba[l][None, :],
            W2[l].T, B2[l][None, :], W3[l].T, B3[l][None, :])


def _ffn_weights(l, LN1w, LN1b, DWin, DBin, DWout, DBout, LN2w, LN2b):
    return (LN1w[l][None, :], LN1b[l][None, :], DWin[l].T, DBin[l][None, :],
            DWout[l].T, DBout[l][None, :], LN2w[l][None, :], LN2b[l][None, :])


def kernel(edge_features, neighbor_indices, mask, W1, B1, W2, B2, W3, B3,
           LN1w, LN1b, DWin, DBin, DWout, DBout, LN2w, LN2b,
           W11, B11, W12, B12, W13, B13, LN3w, LN3b):
    ef2d = edge_features.reshape(B, D)
    mask2d = mask[:, None]
    idx = neighbor_indices.reshape(-1).astype(jnp.int32)
    idx2d = jnp.concatenate(
        [idx, jnp.zeros((B_PAD - B,), jnp.int32)]).reshape(IDX_ROWS, 128)

    init_w = ((W1[0][:, D:2 * D].T, B1[0][None, :], W2[0].T, B2[0][None, :],
               W3[0].T, B3[0][None, :])
              + _ffn_weights(0, LN1w, LN1b, DWin, DBin, DWout, DBout, LN2w, LN2b))
    node = _call_init(ef2d, mask2d, *init_w)

    for l in range(L - 1):
        nbr = _sc_gather(node, idx2d)
        w = (_mlp_weights(l, W11, B11, W12, B12, W13, B13)
             + (LN3w[l][None, :], LN3b[l][None, :])
             + _mlp_weights(l + 1, W1, B1, W2, B2, W3, B3)
             + _ffn_weights(l + 1, LN1w, LN1b, DWin, DBin, DWout, DBout,
                            LN2w, LN2b))
        ef2d, node = _call_fused(ef2d, nbr, node, mask2d, *w)

    nbr = _sc_gather(node, idx2d)
    w = (_mlp_weights(L - 1, W11, B11, W12, B12, W13, B13)
         + (LN3w[L - 1][None, :], LN3b[L - 1][None, :]))
    ef2d = _call_final(ef2d, nbr, node, *w)

    return node, ef2d.reshape(N, K, D)


# speedup vs baseline: 1.0421x; 1.0060x over previous
import functools

import jax
import jax.numpy as jnp
from jax import lax
from jax.experimental import pallas as pl
from jax.experimental.pallas import tpu as pltpu
from jax.experimental.pallas import tpu_sc as plsc

N, K, D, H, L = 10000, 16, 128, 128, 3
EPS = 1e-5
SCALE = 30.0

NB = 400
EB = NB * K
GRID = N // NB

B = N * K
B_PAD = 163840


def _gelu(x):
    return x * (lax.erf(x * 0.7071067811865476) * 0.5 + 0.5)


def _bdot(a, b):
    return jnp.dot(a.astype(jnp.bfloat16), b.astype(jnp.bfloat16),
                   preferred_element_type=jnp.float32)


def _ln(x, w, b):
    m = jnp.mean(x, axis=-1, keepdims=True)
    xc = x - m
    v = jnp.mean(xc * xc, axis=-1, keepdims=True)
    return xc * lax.rsqrt(v + EPS) * w + b


def _tile_nodes(x):
    return jnp.broadcast_to(x[:, None, :], (NB, K, x.shape[-1])).reshape(EB, x.shape[-1])


def _msg_sum(m):
    return jnp.sum(m.reshape(NB, K, m.shape[-1]), axis=1)


def _node_update(node, msum, ln1w, ln1b, dwinT, dbin, dwoutT, dbout, ln2w, ln2b, mask):
    nd = _ln(node + msum * (1.0 / SCALE), ln1w, ln1b)
    d_ = _bdot(_gelu(_bdot(nd, dwinT) + dbin), dwoutT) + dbout
    return _ln(nd + d_, ln2w, ln2b) * mask


def _init_body(ef_ref, mask_ref,
               w1mT, b1, w2T, b2, w3T, b3,
               ln1w, ln1b, dwinT, dbin, dwoutT, dbout, ln2w, ln2b,
               node_out):
    ef = ef_ref[...]
    h = _gelu(_bdot(ef, w1mT[...]) + b1[...])
    h = _gelu(_bdot(h, w2T[...]) + b2[...])
    msum = _bdot(_msg_sum(h), w3T[...]) + float(K) * b3[...]
    node_out[...] = _node_update(
        jnp.zeros((NB, D), jnp.float32), msum,
        ln1w[...], ln1b[...], dwinT[...], dbin[...], dwoutT[...], dbout[...],
        ln2w[...], ln2b[...], mask_ref[...])


def _mlp2(node, ef, nbr, waT, wbT, wcT, b1, w2T, b2):
    hn = _bdot(node, waT)
    h = _gelu(_tile_nodes(hn)
              + _bdot(ef, wbT)
              + _bdot(nbr, wcT) + b1)
    return _gelu(_bdot(h, w2T) + b2)


def _fused_body(ef_ref, nbr_ref, node_ref, mask_ref,
                w11aT, w11bT, w11cT, b11, w12T, b12, w13T, b13, ln3w, ln3b,
                w1aT, w1bT, w1cT, b1, w2T, b2, w3T, b3,
                ln1w, ln1b, dwinT, dbin, dwoutT, dbout, ln2w, ln2b,
                ef_out, node_out):
    ef = ef_ref[...]
    nbr = nbr_ref[...]
    node = node_ref[...]
    h = _mlp2(node, ef, nbr, w11aT[...], w11bT[...], w11cT[...],
              b11[...], w12T[...], b12[...])
    em = _bdot(h, w13T[...]) + b13[...]
    efn = _ln(ef + em, ln3w[...], ln3b[...])
    ef_out[...] = efn
    h = _mlp2(node, efn, nbr, w1aT[...], w1bT[...], w1cT[...],
              b1[...], w2T[...], b2[...])
    msum = _bdot(_msg_sum(h), w3T[...]) + float(K) * b3[...]
    node_out[...] = _node_update(
        node, msum,
        ln1w[...], ln1b[...], dwinT[...], dbin[...], dwoutT[...], dbout[...],
        ln2w[...], ln2b[...], mask_ref[...])


def _final_body(ef_ref, nbr_ref, node_ref,
                w11aT, w11bT, w11cT, b11, w12T, b12, w13T, b13, ln3w, ln3b,
                ef_out):
    ef = ef_ref[...]
    h = _mlp2(node_ref[...], ef, nbr_ref[...],
              w11aT[...], w11bT[...], w11cT[...],
              b11[...], w12T[...], b12[...])
    em = _bdot(h, w13T[...]) + b13[...]
    ef_out[...] = _ln(ef + em, ln3w[...], ln3b[...])


_EDGE_SPEC = pl.BlockSpec((EB, D), lambda i: (i, 0))
_NBR_SPEC = pl.BlockSpec((EB, D), lambda i: (i, 0))
_NODE_SPEC = pl.BlockSpec((NB, D), lambda i: (i, 0))
_MASK_SPEC = pl.BlockSpec((NB, 1), lambda i: (i, 0))


def _w_spec(shape):
    return pl.BlockSpec(shape, lambda i: tuple(0 for _ in shape))


def _wspecs(shapes):
    return [_w_spec(s) for s in shapes]


_MAT = (D, H)
_VEC = (1, H)
_MLP_W_SHAPES = [_MAT, _MAT, _MAT, _VEC, _MAT, _VEC, _MAT, _VEC]
_LN_SHAPES = [_VEC, _VEC]
_FFN_SHAPES = [_VEC, _VEC, _MAT, _VEC, _MAT, _VEC, _VEC, _VEC]

_TC_PARAMS = pltpu.CompilerParams(dimension_semantics=("arbitrary",))


def _call_init(ef2d, mask2d, *weights):
    return pl.pallas_call(
        _init_body,
        grid=(GRID,),
        in_specs=[_EDGE_SPEC, _MASK_SPEC]
        + _wspecs([_MAT, _VEC, _MAT, _VEC, _MAT, _VEC] + _FFN_SHAPES),
        out_specs=_NODE_SPEC,
        out_shape=jax.ShapeDtypeStruct((N, D), jnp.float32),
        compiler_params=_TC_PARAMS,
    )(ef2d, mask2d, *weights)


def _call_fused(ef2d, nbr, node, mask2d, *weights):
    return pl.pallas_call(
        _fused_body,
        grid=(GRID,),
        in_specs=[_EDGE_SPEC, _NBR_SPEC, _NODE_SPEC, _MASK_SPEC]
        + _wspecs(_MLP_W_SHAPES + _LN_SHAPES + _MLP_W_SHAPES + _FFN_SHAPES),
        out_specs=[_EDGE_SPEC, _NODE_SPEC],
        out_shape=[jax.ShapeDtypeStruct((B, D), jnp.float32),
                   jax.ShapeDtypeStruct((N, D), jnp.float32)],
        compiler_params=_TC_PARAMS,
    )(ef2d, nbr, node, mask2d, *weights)


def _call_final(ef2d, nbr, node, *weights):
    return pl.pallas_call(
        _final_body,
        grid=(GRID,),
        in_specs=[_EDGE_SPEC, _NBR_SPEC, _NODE_SPEC]
        + _wspecs(_MLP_W_SHAPES + _LN_SHAPES),
        out_specs=_EDGE_SPEC,
        out_shape=jax.ShapeDtypeStruct((B, D), jnp.float32),
        compiler_params=_TC_PARAMS,
    )(ef2d, nbr, node, *weights)


_CROWS = 64
_NBUF = 8
_PAIR = B_PAD // 16 // _CROWS
_C_FAST = 128
_C_SLOW = _PAIR - _C_FAST
_FAST_CORE = 0


def _sc_gather(node_table, idx2d):
    mesh = plsc.VectorSubcoreMesh(core_axis_name="c", subcore_axis_name="s")

    @functools.partial(
        pl.kernel,
        out_type=jax.ShapeDtypeStruct((B_PAD, D), jnp.float32),
        mesh=mesh,
        scratch_types=[
            pltpu.VMEM((_PAIR, _CROWS), jnp.int32),
            [pltpu.VMEM((_CROWS, D), jnp.float32) for _ in range(_NBUF)],
            [pltpu.SemaphoreType.DMA for _ in range(_NBUF)],
            [pltpu.SemaphoreType.DMA for _ in range(_NBUF)],
        ],
    )
    def gk(table_hbm, idx_hbm, out_hbm, idx_v, bufs, gsems, ssems):
        s = lax.axis_index("s")
        c = lax.axis_index("c")
        is_fast = (c == _FAST_CORE)
        loff = jnp.where(is_fast, 0, _C_FAST)
        ngroups = jnp.where(is_fast, _C_FAST // _NBUF, _C_SLOW // _NBUF)
        pbase = s * _PAIR
        pltpu.sync_copy(idx_hbm.at[pl.ds(pbase, _PAIR)], idx_v)

        for b in range(_NBUF):
            pltpu.async_copy(table_hbm.at[idx_v.at[loff + b]], bufs[b],
                             gsems[b])

        def body(g, carry):
            for b in range(_NBUF):
                i = loff + g * _NBUF + b
                pltpu.make_async_copy(
                    table_hbm.at[idx_v.at[i]], bufs[b], gsems[b]).wait()
                pltpu.async_copy(
                    bufs[b], out_hbm.at[pl.ds((pbase + i) * _CROWS, _CROWS)],
                    ssems[b])

                @pl.when(g < ngroups - 1)
                def _():
                    pltpu.make_async_copy(
                        bufs[b], out_hbm.at[pl.ds(pbase * _CROWS, _CROWS)],
                        ssems[b]).wait()
                    pltpu.async_copy(
                        table_hbm.at[idx_v.at[i + _NBUF]], bufs[b], gsems[b])
            return carry

        lax.fori_loop(0, ngroups, body, 0)
        for b in range(_NBUF):
            pltpu.make_async_copy(
                bufs[b], out_hbm.at[pl.ds(pbase * _CROWS, _CROWS)],
                ssems[b]).wait()

    return gk(node_table, idx2d)


def _mlp_weights(l, Wa, Ba, W2, B2, W3, B3):
    w = Wa[l]
    return (w[:, :D].T, w[:, D:2 * D].T, w[:, 2 * D:].T, Ba[l][None, :],
            W2[l].T, B2[l][None, :], W3[l].T, B3[l][None, :])


def _ffn_weights(l, LN1w, LN1b, DWin, DBin, DWout, DBout, LN2w, LN2b):
    return (LN1w[l][None, :], LN1b[l][None, :], DWin[l].T, DBin[l][None, :],
            DWout[l].T, DBout[l][None, :], LN2w[l][None, :], LN2b[l][None, :])


def kernel(edge_features, neighbor_indices, mask, W1, B1, W2, B2, W3, B3,
           LN1w, LN1b, DWin, DBin, DWout, DBout, LN2w, LN2b,
           W11, B11, W12, B12, W13, B13, LN3w, LN3b):
    ef2d = edge_features.reshape(B, D)
    mask2d = mask[:, None]
    idx = neighbor_indices.reshape(-1).astype(jnp.int32)
    idx2d = jnp.concatenate(
        [idx, jnp.zeros((B_PAD - B,), jnp.int32)]).reshape(-1, _CROWS)

    init_w = ((W1[0][:, D:2 * D].T, B1[0][None, :], W2[0].T, B2[0][None, :],
               W3[0].T, B3[0][None, :])
              + _ffn_weights(0, LN1w, LN1b, DWin, DBin, DWout, DBout, LN2w, LN2b))
    node = _call_init(ef2d, mask2d, *init_w)

    for l in range(L - 1):
        nbr = _sc_gather(node, idx2d)
        w = (_mlp_weights(l, W11, B11, W12, B12, W13, B13)
             + (LN3w[l][None, :], LN3b[l][None, :])
             + _mlp_weights(l + 1, W1, B1, W2, B2, W3, B3)
             + _ffn_weights(l + 1, LN1w, LN1b, DWin, DBin, DWout, DBout,
                            LN2w, LN2b))
        ef2d, node = _call_fused(ef2d, nbr, node, mask2d, *w)

    nbr = _sc_gather(node, idx2d)
    w = (_mlp_weights(L - 1, W11, B11, W12, B12, W13, B13)
         + (LN3w[L - 1][None, :], LN3b[L - 1][None, :]))
    ef2d = _call_final(ef2d, nbr, node, *w)

    return node, ef2d.reshape(N, K, D)
